# Initial kernel scaffold; baseline (speedup 1.0000x reference)
import jax
import jax.numpy as jnp
from jax.experimental import pallas as pl


def kernel(b, N, K, L, gpn_obj_ind, gpn_pred_ind, gpn_nrel_ind, gpn_pool_mtx, att_feats, x_pred, fc_feats, att_masks, W1, b1, W2, b2, W3, b3, W4, b4):
    raise NotImplementedError("write your pallas kernel here")


# speedup vs baseline: 4.0614x; 4.0614x over previous
import functools
import jax
import jax.numpy as jnp
from jax import lax
from jax.experimental import pallas as pl
from jax.experimental.pallas import tpu as pltpu

B_, S_, N_, M_, P_, L_, HID_ = 16, 64, 36, 100, 256, 1024, 512
SC = 8
MPAD = 128


def _k1_body(idx_ref, pool_ref, att_ref, w1_ref, w2p_ref, prm_ref,
             score_ref, smax_ref, best_ref, bidx_ref):
    sc = pl.program_id(1)
    idx = idx_ref[0]
    att = att_ref[0]
    iota_m = lax.broadcasted_iota(jnp.int32, (1, MPAD), 1)
    a_rows = []
    for s in range(SC):
        oh = (idx[s][:, None] == iota_m).astype(jnp.float32)
        a_rows.append(jnp.dot(pool_ref[s, 0].astype(jnp.bfloat16),
                              oh.astype(jnp.bfloat16),
                              preferred_element_type=jnp.float32))
    a_mat = jnp.concatenate(a_rows, axis=0)
    clean = jnp.dot(a_mat.astype(jnp.bfloat16), att.astype(jnp.bfloat16),
                    preferred_element_type=jnp.float32)
    mx, av = [], []
    for s in range(SC):
        blk = clean[s * N_:(s + 1) * N_]
        mx.append(jnp.max(blk, axis=0, keepdims=True))
        av.append(jnp.mean(blk, axis=0, keepdims=True))
    read_out = jnp.concatenate(
        [jnp.concatenate(mx, axis=0), jnp.concatenate(av, axis=0)], axis=1)
    h = jnp.maximum(
        jnp.dot(read_out.astype(jnp.bfloat16), w1_ref[...].astype(jnp.bfloat16),
                preferred_element_type=jnp.float32)
        + prm_ref[0:1, :], 0.0)
    logit = jnp.dot(h.astype(jnp.bfloat16), w2p_ref[...].astype(jnp.bfloat16),
                    preferred_element_type=jnp.float32)[:, 0] + prm_ref[2, 0]
    score = jax.nn.sigmoid(logit)
    score_ref[0, 0, :] = score
    gidx = lax.broadcasted_iota(jnp.int32, (1, SC), 1)[0] + sc * SC

    @pl.when(sc == 0)
    def _():
        best_ref[0, :] = score
        bidx_ref[0, :] = gidx

    @pl.when(sc > 0)
    def _():
        better = score > best_ref[0, :]
        best_ref[0, :] = jnp.where(better, score, best_ref[0, :])
        bidx_ref[0, :] = jnp.where(better, gidx, bidx_ref[0, :])

    @pl.when(sc == (S_ // SC) - 1)
    def _():
        best = best_ref[0, :]
        amax = jnp.max(best)
        cand = jnp.where(best == amax, bidx_ref[0, :], S_)
        smax_ref[0, 0, :] = jnp.broadcast_to(jnp.min(cand), (8,))


def _k3_body(smax_ref, idx_ref, pool_ref, att_ref, masks_ref,
             w3_ref, w4_ref, prm34_ref,
             attout_ref, fc_ref, maskout_ref):
    smax = smax_ref[0, 0, 0]
    sel_col = lax.broadcasted_iota(jnp.int32, (S_, 1), 0) == smax
    idx = jnp.sum(jnp.where(sel_col, idx_ref[0], 0), axis=0)
    maskout_ref[0, 0, :] = jnp.sum(
        jnp.where(sel_col, masks_ref[0], 0.0), axis=0)
    pool = jnp.sum(jnp.where(sel_col[:, :, None], pool_ref[:, 0], 0.0),
                   axis=0)
    iota_m = lax.broadcasted_iota(jnp.int32, (1, MPAD), 1)
    oh = (idx[:, None] == iota_m).astype(jnp.float32)
    g = jnp.dot(oh, att_ref[0], preferred_element_type=jnp.float32, precision=lax.Precision.HIGHEST)
    attout_ref[0] = g
    clean = jnp.dot(pool.astype(jnp.bfloat16), g.astype(jnp.bfloat16),
                    preferred_element_type=jnp.float32)
    mx = jnp.max(clean, axis=0, keepdims=True)
    av = jnp.mean(clean, axis=0, keepdims=True)
    ro = jnp.concatenate([mx, av], axis=1)
    h = jnp.dot(ro.astype(jnp.bfloat16), w3_ref[...].astype(jnp.bfloat16),
                preferred_element_type=jnp.float32) \
        + prm34_ref[0:1, :HID_]
    fc = jnp.dot(h.astype(jnp.bfloat16), w4_ref[...].astype(jnp.bfloat16),
                 preferred_element_type=jnp.float32) \
        + prm34_ref[1:2, :]
    fc_ref[0] = fc


@jax.jit
def _run(gpn_obj_ind, gpn_pool_mtx, att_feats, att_masks,
         W1, b1, W2, b2, W3, b3, W4, b4):
    L2 = 2 * L_
    obj = gpn_obj_ind.astype(jnp.int32)
    att_pad = jnp.pad(att_feats, ((0, 0), (0, MPAD - M_), (0, 0)))
    prm = jnp.concatenate(
        [b1[None, :], W2.T, jnp.broadcast_to(b2, (1, HID_))], axis=0)
    w2p = jnp.pad(W2, ((0, 0), (0, 127)))

    scores3, smax3 = pl.pallas_call(
        _k1_body,
        grid=(B_, S_ // SC),
        in_specs=[
            pl.BlockSpec((1, SC, N_), lambda b, sc: (b, sc, 0)),
            pl.BlockSpec((SC, 1, N_, N_), lambda b, sc: (sc, b, 0, 0)),
            pl.BlockSpec((1, MPAD, L_), lambda b, sc: (b, 0, 0)),
            pl.BlockSpec((L2, HID_), lambda b, sc: (0, 0)),
            pl.BlockSpec((HID_, 128), lambda b, sc: (0, 0)),
            pl.BlockSpec((3, HID_), lambda b, sc: (0, 0)),
        ],
        out_specs=[
            pl.BlockSpec((1, 1, SC), lambda b, sc: (b * (S_ // SC) + sc, 0, 0)),
            pl.BlockSpec((1, 1, 8), lambda b, sc: (b, 0, 0)),
        ],
        out_shape=[
            jax.ShapeDtypeStruct((B_ * S_ // SC, 1, SC), jnp.float32),
            jax.ShapeDtypeStruct((B_, 1, 8), jnp.int32),
        ],
        scratch_shapes=[pltpu.VMEM((1, SC), jnp.float32),
                        pltpu.VMEM((1, SC), jnp.int32)],
    )(obj, gpn_pool_mtx, att_pad, W1, w2p, prm)

    subgraph_score = scores3.reshape(B_, S_)
    sub_max_ind = smax3[:, 0, 0]

    b3p = jnp.pad(b3, (0, L2 - HID_))
    prm34 = jnp.concatenate([b3p[None, :], b4[None, :]], axis=0)

    att_out, fc_out, mask_out = pl.pallas_call(
        _k3_body,
        grid=(B_,),
        in_specs=[
            pl.BlockSpec((1, 1, 8), lambda b: (b, 0, 0)),
            pl.BlockSpec((1, S_, N_), lambda b: (b, 0, 0)),
            pl.BlockSpec((S_, 1, N_, N_), lambda b: (0, b, 0, 0)),
            pl.BlockSpec((1, MPAD, L_), lambda b: (b, 0, 0)),
            pl.BlockSpec((1, S_, N_), lambda b: (b, 0, 0)),
            pl.BlockSpec((L2, HID_), lambda b: (0, 0)),
            pl.BlockSpec((HID_, L2), lambda b: (0, 0)),
            pl.BlockSpec((2, L2), lambda b: (0, 0)),
        ],
        out_specs=[
            pl.BlockSpec((1, N_, L_), lambda b: (b, 0, 0)),
            pl.BlockSpec((1, 1, L2), lambda b: (b, 0, 0)),
            pl.BlockSpec((1, 1, N_), lambda b: (b, 0, 0)),
        ],
        out_shape=[
            jax.ShapeDtypeStruct((B_, N_, L_), jnp.float32),
            jax.ShapeDtypeStruct((B_, 1, L2), jnp.float32),
            jax.ShapeDtypeStruct((B_, 1, N_), jnp.float32),
        ],
    )(smax3, obj, gpn_pool_mtx, att_pad, att_masks, W3, W4, prm34)

    return (sub_max_ind, subgraph_score, att_out,
            fc_out.reshape(B_, L2), mask_out.reshape(B_, N_))


def kernel(b, N, K, L, gpn_obj_ind, gpn_pred_ind, gpn_nrel_ind, gpn_pool_mtx,
           att_feats, x_pred, fc_feats, att_masks,
           W1, b1, W2, b2, W3, b3, W4, b4):
    sub_max_ind, subgraph_score, att_out, fc_out, mask_out = _run(
        gpn_obj_ind, gpn_pool_mtx, att_feats, att_masks,
        W1, b1, W2, b2, W3, b3, W4, b4)
    return (sub_max_ind, gpn_obj_ind, subgraph_score, att_out, fc_out, mask_out)
